# baseline (speedup 1.0000x reference)
import jax
import jax.numpy as jnp
from jax import lax
from jax.experimental import pallas as pl
from jax.experimental.pallas import tpu as pltpu
from jax.experimental.pallas import tpu_sc as plsc

N_ATOMS = 100000
D = 128
V_ROWS = 100
NW = 32
PER_W = 3136
C = 256
SIZES = (256,) * 12 + (64,)
OFFS = tuple(256 * i for i in range(13))
NCH = len(SIZES)
W31_ROWS = N_ATOMS - (NW - 1) * PER_W


def _gather_body(z_hbm, emb_hbm, out_hbm,
                 table_v, idx_v, buf0, buf1, buf2, gs0, gs1, ws0, ws1, ws2):
    bufs, wsems = (buf0, buf1, buf2), (ws0, ws1, ws2)
    wid = lax.axis_index("s") * 2 + lax.axis_index("c")
    base = wid * PER_W
    th = pltpu.async_copy(emb_hbm, table_v, gs0)

    @pl.when(wid < NW - 1)
    def _load_idx_full():
        pltpu.async_copy(z_hbm.at[pl.ds(base, PER_W)],
                         idx_v.at[pl.ds(0, PER_W)], gs1).wait()

    @pl.when(wid == NW - 1)
    def _load_idx_tail():
        pltpu.async_copy(z_hbm.at[pl.ds(base, W31_ROWS)],
                         idx_v.at[pl.ds(0, W31_ROWS)], gs1).wait()
        zeros = jnp.zeros((16,), jnp.int32)
        for t in range(W31_ROWS, PER_W + 16, 16):
            idx_v[pl.ds(t, 16)] = zeros

    th.wait()

    wh = [None, None, None]
    for k in range(NCH):
        b = k % 3
        off, n = OFFS[k], SIZES[k]
        if wh[b] is not None:
            wh[b].wait()
            wh[b] = None

        _buf = bufs[b]
        _off = off

        @plsc.parallel_loop(0, n, unroll=2)
        def _row(i):
            zv = idx_v[pl.ds(_off + i, 16)]
            s = zv[0] * D
            dbase = i * D
            for t in range(0, D, 16):
                _buf[pl.ds(dbase + t, 16)] = table_v[pl.ds(s + t, 16)]

        w31 = min(max(W31_ROWS - off, 0), n)
        src_full = bufs[b] if n == C else bufs[b].at[pl.ds(0, n * D)]
        dst_full = out_hbm.at[pl.ds((base + off) * D, n * D)]
        if w31 == n:
            wh[b] = pltpu.async_copy(src_full, dst_full, wsems[b])
        else:
            @pl.when(wid < NW - 1)
            def _full():
                pltpu.sync_copy(src_full, dst_full)

            if w31 > 0:
                @pl.when(wid == NW - 1)
                def _tail():
                    pltpu.sync_copy(
                        bufs[b].at[pl.ds(0, w31 * D)],
                        out_hbm.at[pl.ds((base + off) * D, w31 * D)])
    for h in wh:
        if h is not None:
            h.wait()


def kernel(dR, Z, embeddings):
    del dR
    emb_flat = embeddings.reshape(-1)
    mesh = plsc.VectorSubcoreMesh(core_axis_name="c", subcore_axis_name="s")
    f = pl.kernel(
        _gather_body,
        out_type=jax.ShapeDtypeStruct((N_ATOMS * D,), jnp.float32),
        mesh=mesh,
        scratch_types=[
            pltpu.VMEM((V_ROWS * D,), jnp.float32),
            pltpu.VMEM((PER_W + 16,), jnp.int32),
            pltpu.VMEM((C * D,), jnp.float32),
            pltpu.VMEM((C * D,), jnp.float32),
            pltpu.VMEM((C * D,), jnp.float32),
            pltpu.SemaphoreType.DMA,
            pltpu.SemaphoreType.DMA,
            pltpu.SemaphoreType.DMA,
            pltpu.SemaphoreType.DMA,
            pltpu.SemaphoreType.DMA,
        ],
        compiler_params=pltpu.CompilerParams(needs_layout_passes=False),
    )
    return f(Z, emb_flat).reshape(N_ATOMS, D)


# speedup vs baseline: 1.0442x; 1.0442x over previous
import jax
import jax.numpy as jnp
from jax import lax
from jax.experimental import pallas as pl
from jax.experimental.pallas import tpu as pltpu
from jax.experimental.pallas import tpu_sc as plsc

N_ATOMS = 100000
D = 128
V_ROWS = 100
NW = 32
PER_W = 3136
C = 384
SIZES = (384, 384, 384, 384, 384, 384, 384, 384, 64)
OFFS = tuple(384 * i for i in range(9))
NCH = len(SIZES)
W31_ROWS = N_ATOMS - (NW - 1) * PER_W


def _gather_body(z_hbm, emb_hbm, out_hbm,
                 table_v, idx_v, buf0, buf1, gs0, gs1, ws0, ws1):
    bufs, wsems = (buf0, buf1), (ws0, ws1)
    wid = lax.axis_index("s") * 2 + lax.axis_index("c")
    base = wid * PER_W
    th = pltpu.async_copy(emb_hbm, table_v, gs0)

    @pl.when(wid < NW - 1)
    def _load_idx_full():
        pltpu.async_copy(z_hbm.at[pl.ds(base, PER_W)],
                         idx_v.at[pl.ds(0, PER_W)], gs1).wait()

    @pl.when(wid == NW - 1)
    def _load_idx_tail():
        pltpu.async_copy(z_hbm.at[pl.ds(base, W31_ROWS)],
                         idx_v.at[pl.ds(0, W31_ROWS)], gs1).wait()
        zeros = jnp.zeros((16,), jnp.int32)
        for t in range(W31_ROWS, PER_W + 16, 16):
            idx_v[pl.ds(t, 16)] = zeros

    th.wait()

    wh = [None, None]
    for k in range(NCH):
        b = k % 2
        off, n = OFFS[k], SIZES[k]
        if wh[b] is not None:
            wh[b].wait()
            wh[b] = None

        _buf = bufs[b]
        _off = off

        @plsc.parallel_loop(0, n, unroll=2)
        def _row(i):
            zv = idx_v[pl.ds(_off + i, 16)]
            s = zv[0] * D
            dbase = i * D
            for t in range(0, D, 16):
                _buf[pl.ds(dbase + t, 16)] = table_v[pl.ds(s + t, 16)]

        w31 = min(max(W31_ROWS - off, 0), n)
        src_full = bufs[b] if n == C else bufs[b].at[pl.ds(0, n * D)]
        dst_full = out_hbm.at[pl.ds((base + off) * D, n * D)]
        if w31 == n:
            wh[b] = pltpu.async_copy(src_full, dst_full, wsems[b])
        else:
            @pl.when(wid < NW - 1)
            def _full():
                pltpu.sync_copy(src_full, dst_full)

            if w31 > 0:
                @pl.when(wid == NW - 1)
                def _tail():
                    pltpu.sync_copy(
                        bufs[b].at[pl.ds(0, w31 * D)],
                        out_hbm.at[pl.ds((base + off) * D, w31 * D)])
    for h in wh:
        if h is not None:
            h.wait()


def kernel(dR, Z, embeddings):
    del dR
    emb_flat = embeddings.reshape(-1)
    mesh = plsc.VectorSubcoreMesh(core_axis_name="c", subcore_axis_name="s")
    f = pl.kernel(
        _gather_body,
        out_type=jax.ShapeDtypeStruct((N_ATOMS * D,), jnp.float32),
        mesh=mesh,
        scratch_types=[
            pltpu.VMEM((V_ROWS * D,), jnp.float32),
            pltpu.VMEM((PER_W + 16,), jnp.int32),
            pltpu.VMEM((C * D,), jnp.float32),
            pltpu.VMEM((C * D,), jnp.float32),
            pltpu.SemaphoreType.DMA,
            pltpu.SemaphoreType.DMA,
            pltpu.SemaphoreType.DMA,
            pltpu.SemaphoreType.DMA,
        ],
        compiler_params=pltpu.CompilerParams(needs_layout_passes=False),
    )
    return f(Z, emb_flat).reshape(N_ATOMS, D)


# speedup vs baseline: 1.0450x; 1.0007x over previous
import jax
import jax.numpy as jnp
from jax import lax
from jax.experimental import pallas as pl
from jax.experimental.pallas import tpu as pltpu
from jax.experimental.pallas import tpu_sc as plsc

N_ATOMS = 100000
D = 128
V_ROWS = 100
NW = 32
PER_W = 3136
C = 384
SIZES = (384, 384, 384, 384, 384, 384, 384, 384, 64)
OFFS = tuple(384 * i for i in range(9))
NCH = len(SIZES)
W31_ROWS = N_ATOMS - (NW - 1) * PER_W


def _gather_body(z_hbm, emb_hbm, out_hbm,
                 table_v, idx_v, buf0, buf1, gs0, gs1, ws0, ws1):
    bufs, wsems = (buf0, buf1), (ws0, ws1)
    wid = lax.axis_index("s") * 2 + lax.axis_index("c")
    base = wid * PER_W
    th = pltpu.async_copy(emb_hbm, table_v, gs0)

    @pl.when(wid < NW - 1)
    def _load_idx_full():
        pltpu.async_copy(z_hbm.at[pl.ds(base, PER_W)],
                         idx_v.at[pl.ds(0, PER_W)], gs1).wait()

    @pl.when(wid == NW - 1)
    def _load_idx_tail():
        pltpu.async_copy(z_hbm.at[pl.ds(base, W31_ROWS)],
                         idx_v.at[pl.ds(0, W31_ROWS)], gs1).wait()
        zeros = jnp.zeros((16,), jnp.int32)
        for t in range(W31_ROWS, PER_W + 16, 16):
            idx_v[pl.ds(t, 16)] = zeros

    th.wait()

    wh = [None, None]
    for k in range(NCH):
        b = k % 2
        off, n = OFFS[k], SIZES[k]
        if wh[b] is not None:
            wh[b].wait()
            wh[b] = None

        _buf = bufs[b]
        _off = off

        @plsc.parallel_loop(0, n, unroll=2)
        def _row(i):
            zv = idx_v[pl.ds(_off + i, 16)]
            s = zv[0] * D
            dbase = i * D
            for t in range(0, D, 16):
                _buf[pl.ds(dbase + t, 16)] = table_v[pl.ds(s + t, 16)]

        w31 = min(max(W31_ROWS - off, 0), n)
        src_full = bufs[b] if n == C else bufs[b].at[pl.ds(0, n * D)]
        dst_full = out_hbm.at[pl.ds((base + off) * D, n * D)]
        if w31 == n:
            wh[b] = pltpu.async_copy(src_full, dst_full, wsems[b])
        else:
            @pl.when(wid < NW - 1)
            def _full():
                pltpu.sync_copy(src_full, dst_full)

            if w31 > 0:
                @pl.when(wid == NW - 1)
                def _tail():
                    pltpu.sync_copy(
                        bufs[b].at[pl.ds(0, w31 * D)],
                        out_hbm.at[pl.ds((base + off) * D, w31 * D)])
    for h in wh:
        if h is not None:
            h.wait()


def kernel(dR, Z, embeddings):
    del dR
    emb_flat = embeddings.reshape(-1)
    mesh = plsc.VectorSubcoreMesh(core_axis_name="c", subcore_axis_name="s")
    f = pl.kernel(
        _gather_body,
        out_type=jax.ShapeDtypeStruct((N_ATOMS * D,), jnp.float32),
        mesh=mesh,
        scratch_types=[
            pltpu.VMEM((V_ROWS * D,), jnp.float32),
            pltpu.VMEM((PER_W + 16,), jnp.int32),
            pltpu.VMEM((C * D,), jnp.float32),
            pltpu.VMEM((C * D,), jnp.float32),
            pltpu.SemaphoreType.DMA,
            pltpu.SemaphoreType.DMA,
            pltpu.SemaphoreType.DMA,
            pltpu.SemaphoreType.DMA,
        ],
        compiler_params=pltpu.CompilerParams(
            needs_layout_passes=False, skip_device_barrier=True),
    )
    return f(Z, emb_flat).reshape(N_ATOMS, D)
